# Initial kernel scaffold; baseline (speedup 1.0000x reference)
import jax
import jax.numpy as jnp
from jax.experimental import pallas as pl


def kernel(nfeat, efeat, timestamp, basis_freq, phase, Wt, bt, Ws1, bs1, Wn1, bn1, Ws2, bs2, Wn2, bn2, dst_ids, src_max_eid):
    raise NotImplementedError("write your pallas kernel here")


# speedup vs baseline: 1.2907x; 1.2907x over previous
import functools

import jax
import jax.numpy as jnp
from jax import lax
from jax.experimental import pallas as pl
from jax.experimental.pallas import tpu as pltpu
from jax.experimental.pallas import tpu_sc as plsc

_NC = 2
_NS = 16
_NW = _NC * _NS

_PREC = lax.Precision.HIGHEST


def _sc_gather(table, idx2d, chunk):
  nch = idx2d.shape[0]
  d = table.shape[1]
  nk = (nch + _NW - 1) // _NW
  mesh = plsc.VectorSubcoreMesh(
      core_axis_name="c", subcore_axis_name="s",
      num_cores=_NC, num_subcores=_NS)

  @functools.partial(
      pl.kernel,
      out_type=jax.ShapeDtypeStruct((nch * chunk, d), table.dtype),
      mesh=mesh,
      scratch_types=[
          pltpu.VMEM((chunk,), jnp.int32),
          pltpu.VMEM((chunk, d), table.dtype),
          pltpu.SemaphoreType.DMA,
      ],
  )
  def gather_kernel(table_hbm, idx_hbm, out_hbm, idx_v, rows_v, sem):
    w = lax.axis_index("s") * _NC + lax.axis_index("c")

    def body(k, _):
      cid = k * _NW + w

      @pl.when(cid < nch)
      def _():
        pltpu.sync_copy(idx_hbm.at[cid], idx_v)
        pltpu.async_copy(table_hbm.at[idx_v], rows_v, sem).wait()
        pltpu.sync_copy(rows_v, out_hbm.at[pl.ds(cid * chunk, chunk)])

      return 0

    lax.fori_loop(0, nk, body, 0, unroll=False)

  return gather_kernel(table, idx2d)


def _tc_project(nfeat, w):
  def body(nf_ref, w_ref, out_ref):
    out_ref[...] = jnp.dot(nf_ref[...], w_ref[...], precision=_PREC,
                           preferred_element_type=jnp.float32)

  return pl.pallas_call(
      body,
      out_shape=jax.ShapeDtypeStruct((nfeat.shape[0], w.shape[1]), jnp.float32),
  )(nfeat, w)


def _tc_encode(g0, efeat, ts, w_e, w_t, bt, bf, ph, block):
  e = g0.shape[0]
  h = w_t.shape[1]
  grid = e // block

  def body(g_ref, ef_ref, ts_ref, we_ref, wt_ref, bt_ref, bf_ref, ph_ref, out_ref):
    te = jnp.cos(ts_ref[...] * bf_ref[...] + ph_ref[...])
    acc = g_ref[...] + bt_ref[...]
    acc += jnp.dot(ef_ref[...], we_ref[...], precision=_PREC,
                   preferred_element_type=jnp.float32)
    acc += jnp.dot(te, wt_ref[...], precision=_PREC,
                   preferred_element_type=jnp.float32)
    out_ref[...] = jnp.maximum(acc, 0.0)

  return pl.pallas_call(
      body,
      grid=(grid,),
      in_specs=[
          pl.BlockSpec((block, h), lambda i: (i, 0)),
          pl.BlockSpec((block, efeat.shape[1]), lambda i: (i, 0)),
          pl.BlockSpec((block, 1), lambda i: (i, 0)),
          pl.BlockSpec(w_e.shape, lambda i: (0, 0)),
          pl.BlockSpec(w_t.shape, lambda i: (0, 0)),
          pl.BlockSpec((1, h), lambda i: (0, 0)),
          pl.BlockSpec((1, h), lambda i: (0, 0)),
          pl.BlockSpec((1, h), lambda i: (0, 0)),
      ],
      out_specs=pl.BlockSpec((block, h), lambda i: (i, 0)),
      out_shape=jax.ShapeDtypeStruct((e, h), jnp.float32),
      compiler_params=pltpu.CompilerParams(
          dimension_semantics=("parallel",)),
  )(g0, efeat, ts, w_e, w_t, bt, bf, ph)


def _tc_sage_layer(dst, gsrc, seg_col, seg_row, ws, wn, b, block):
  e, h = dst.shape
  grid = e // block

  def body(dst_ref, gs_ref, sc_ref, sr_ref, ws_ref, wn_ref, b_ref, out_ref,
           carry_sum, carry_cnt, carry_seg):
    @pl.when(pl.program_id(0) == 0)
    def _():
      carry_seg[0] = -1
      carry_cnt[0] = 0.0
      carry_sum[...] = jnp.zeros_like(carry_sum)

    seg_c = sc_ref[...]
    seg_r = sr_ref[0]
    rowid = lax.broadcasted_iota(jnp.int32, (block, block), 0)
    colid = lax.broadcasted_iota(jnp.int32, (block, block), 1)
    mask = (colid <= rowid) & (seg_c == seg_r)
    mf = mask.astype(jnp.float32)
    cs = jnp.dot(mf, gs_ref[...], precision=_PREC,
                 preferred_element_type=jnp.float32)
    cntl = jnp.sum(mf, axis=1, keepdims=True)

    from_carry = (seg_c == carry_seg[0]).astype(jnp.float32)
    total = cs + from_carry * carry_sum[...]
    cnt = cntl + from_carry * carry_cnt[0]
    agg = total / cnt

    carry_sum[...] = total[block - 1:block, :]
    carry_cnt[0] = cnt[block - 1, 0]
    carry_seg[0] = seg_c[block - 1, 0]

    acc = jnp.dot(dst_ref[...], ws_ref[...], precision=_PREC,
                  preferred_element_type=jnp.float32)
    acc += jnp.dot(agg, wn_ref[...], precision=_PREC,
                   preferred_element_type=jnp.float32)
    out_ref[...] = jnp.maximum(acc + b_ref[...], 0.0)

  return pl.pallas_call(
      body,
      grid=(grid,),
      in_specs=[
          pl.BlockSpec((block, h), lambda i: (i, 0)),
          pl.BlockSpec((block, h), lambda i: (i, 0)),
          pl.BlockSpec((block, 1), lambda i: (i, 0)),
          pl.BlockSpec((1, 1, block), lambda i: (i, 0, 0)),
          pl.BlockSpec((h, h), lambda i: (0, 0)),
          pl.BlockSpec((h, h), lambda i: (0, 0)),
          pl.BlockSpec((1, h), lambda i: (0, 0)),
      ],
      out_specs=pl.BlockSpec((block, h), lambda i: (i, 0)),
      out_shape=jax.ShapeDtypeStruct((e, h), jnp.float32),
      scratch_shapes=[
          pltpu.VMEM((1, h), jnp.float32),
          pltpu.SMEM((1,), jnp.float32),
          pltpu.SMEM((1,), jnp.int32),
      ],
      compiler_params=pltpu.CompilerParams(
          dimension_semantics=("arbitrary",)),
  )(dst, gsrc, seg_col, seg_row, ws, wn, b)


def kernel(nfeat, efeat, timestamp, basis_freq, phase, Wt, bt,
           Ws1, bs1, Wn1, bn1, Ws2, bs2, Wn2, bn2, dst_ids, src_max_eid):
  e = efeat.shape[0]
  f = nfeat.shape[1]
  h = Wt.shape[1]
  ef = efeat.shape[1]

  gchunk = 128
  dst_i = dst_ids.astype(jnp.int32)
  src_i = src_max_eid.astype(jnp.int32)
  dst_idx2d = dst_i.reshape(e // gchunk, gchunk)
  src_idx2d = src_i.reshape(e // gchunk, gchunk)

  wt_u = Wt[:f]
  wt_e = Wt[f:f + ef]
  wt_t = Wt[f + ef:]

  proj = _tc_project(nfeat, wt_u)
  g0 = _sc_gather(proj, dst_idx2d, gchunk)

  ts2 = timestamp.reshape(e, 1)
  dst0 = _tc_encode(g0, efeat, ts2, wt_e, wt_t,
                    bt.reshape(1, h), basis_freq.reshape(1, h),
                    phase.reshape(1, h), block=2000)

  sblk = 256
  seg_col = dst_i.reshape(e, 1)
  seg_row = dst_i.reshape(e // sblk, 1, sblk)

  gsrc0 = _sc_gather(dst0, src_idx2d, gchunk)
  dst1 = _tc_sage_layer(dst0, gsrc0, seg_col, seg_row,
                        Ws1, Wn1, (bs1 + bn1).reshape(1, h), sblk)
  gsrc1 = _sc_gather(dst1, src_idx2d, gchunk)
  dst2 = _tc_sage_layer(dst1, gsrc1, seg_col, seg_row,
                        Ws2, Wn2, (bs2 + bn2).reshape(1, h), sblk)
  src = _sc_gather(dst2, src_idx2d, gchunk)
  return (src, dst2)


# speedup vs baseline: 1.7050x; 1.3210x over previous
import functools

import jax
import jax.numpy as jnp
from jax import lax
from jax.experimental import pallas as pl
from jax.experimental.pallas import tpu as pltpu
from jax.experimental.pallas import tpu_sc as plsc

_NC = 2
_NS = 16
_NW = _NC * _NS

_PREC = lax.Precision.DEFAULT


def _sc_gather(table, idx2d, chunk):
  nch = idx2d.shape[0]
  d = table.shape[1]
  nk = (nch + _NW - 1) // _NW
  mesh = plsc.VectorSubcoreMesh(
      core_axis_name="c", subcore_axis_name="s",
      num_cores=_NC, num_subcores=_NS)

  @functools.partial(
      pl.kernel,
      out_type=jax.ShapeDtypeStruct((nch * chunk, d), table.dtype),
      mesh=mesh,
      scratch_types=[
          pltpu.VMEM((chunk,), jnp.int32),
          pltpu.VMEM((chunk, d), table.dtype),
          pltpu.SemaphoreType.DMA,
      ],
  )
  def gather_kernel(table_hbm, idx_hbm, out_hbm, idx_v, rows_v, sem):
    w = lax.axis_index("s") * _NC + lax.axis_index("c")

    def body(k, _):
      cid = k * _NW + w

      @pl.when(cid < nch)
      def _():
        pltpu.sync_copy(idx_hbm.at[cid], idx_v)
        pltpu.async_copy(table_hbm.at[idx_v], rows_v, sem).wait()
        pltpu.sync_copy(rows_v, out_hbm.at[pl.ds(cid * chunk, chunk)])

      return 0

    lax.fori_loop(0, nk, body, 0, unroll=False)

  return gather_kernel(table, idx2d)


def _tc_project(nfeat, w):
  def body(nf_ref, w_ref, out_ref):
    out_ref[...] = jnp.dot(nf_ref[...], w_ref[...], precision=_PREC,
                           preferred_element_type=jnp.float32)

  return pl.pallas_call(
      body,
      out_shape=jax.ShapeDtypeStruct((nfeat.shape[0], w.shape[1]), jnp.float32),
  )(nfeat, w)


def _tc_encode(g0, efeat, ts, w_e, w_t, bt, bf, ph, block):
  e = g0.shape[0]
  h = w_t.shape[1]
  grid = e // block

  def body(g_ref, ef_ref, ts_ref, we_ref, wt_ref, bt_ref, bf_ref, ph_ref, out_ref):
    te = jnp.cos(ts_ref[...] * bf_ref[...] + ph_ref[...])
    acc = g_ref[...] + bt_ref[...]
    acc += jnp.dot(ef_ref[...], we_ref[...], precision=_PREC,
                   preferred_element_type=jnp.float32)
    acc += jnp.dot(te, wt_ref[...], precision=_PREC,
                   preferred_element_type=jnp.float32)
    out_ref[...] = jnp.maximum(acc, 0.0)

  return pl.pallas_call(
      body,
      grid=(grid,),
      in_specs=[
          pl.BlockSpec((block, h), lambda i: (i, 0)),
          pl.BlockSpec((block, efeat.shape[1]), lambda i: (i, 0)),
          pl.BlockSpec((block, 1), lambda i: (i, 0)),
          pl.BlockSpec(w_e.shape, lambda i: (0, 0)),
          pl.BlockSpec(w_t.shape, lambda i: (0, 0)),
          pl.BlockSpec((1, h), lambda i: (0, 0)),
          pl.BlockSpec((1, h), lambda i: (0, 0)),
          pl.BlockSpec((1, h), lambda i: (0, 0)),
      ],
      out_specs=pl.BlockSpec((block, h), lambda i: (i, 0)),
      out_shape=jax.ShapeDtypeStruct((e, h), jnp.float32),
      compiler_params=pltpu.CompilerParams(
          dimension_semantics=("parallel",)),
  )(g0, efeat, ts, w_e, w_t, bt, bf, ph)


def _tc_sage_layer(dst, gsrc, seg_col, seg_row, ws, wn, b, block):
  e, h = dst.shape
  grid = e // block

  def body(dst_ref, gs_ref, sc_ref, sr_ref, ws_ref, wn_ref, b_ref, out_ref,
           carry_sum, carry_cnt, carry_seg):
    @pl.when(pl.program_id(0) == 0)
    def _():
      carry_seg[0] = -1
      carry_cnt[0] = 0.0
      carry_sum[...] = jnp.zeros_like(carry_sum)

    seg_c = sc_ref[...]
    seg_r = sr_ref[0]
    rowid = lax.broadcasted_iota(jnp.int32, (block, block), 0)
    colid = lax.broadcasted_iota(jnp.int32, (block, block), 1)
    mask = (colid <= rowid) & (seg_c == seg_r)
    mf = mask.astype(jnp.float32)
    cs = jnp.dot(mf, gs_ref[...], precision=_PREC,
                 preferred_element_type=jnp.float32)
    cntl = jnp.sum(mf, axis=1, keepdims=True)

    from_carry = (seg_c == carry_seg[0]).astype(jnp.float32)
    total = cs + from_carry * carry_sum[...]
    cnt = cntl + from_carry * carry_cnt[0]
    agg = total / cnt

    carry_sum[...] = total[block - 1:block, :]
    carry_cnt[0] = cnt[block - 1, 0]
    carry_seg[0] = seg_c[block - 1, 0]

    acc = jnp.dot(dst_ref[...], ws_ref[...], precision=_PREC,
                  preferred_element_type=jnp.float32)
    acc += jnp.dot(agg, wn_ref[...], precision=_PREC,
                   preferred_element_type=jnp.float32)
    out_ref[...] = jnp.maximum(acc + b_ref[...], 0.0)

  return pl.pallas_call(
      body,
      grid=(grid,),
      in_specs=[
          pl.BlockSpec((block, h), lambda i: (i, 0)),
          pl.BlockSpec((block, h), lambda i: (i, 0)),
          pl.BlockSpec((block, 1), lambda i: (i, 0)),
          pl.BlockSpec((1, 1, block), lambda i: (i, 0, 0)),
          pl.BlockSpec((h, h), lambda i: (0, 0)),
          pl.BlockSpec((h, h), lambda i: (0, 0)),
          pl.BlockSpec((1, h), lambda i: (0, 0)),
      ],
      out_specs=pl.BlockSpec((block, h), lambda i: (i, 0)),
      out_shape=jax.ShapeDtypeStruct((e, h), jnp.float32),
      scratch_shapes=[
          pltpu.VMEM((1, h), jnp.float32),
          pltpu.SMEM((1,), jnp.float32),
          pltpu.SMEM((1,), jnp.int32),
      ],
      compiler_params=pltpu.CompilerParams(
          dimension_semantics=("arbitrary",)),
  )(dst, gsrc, seg_col, seg_row, ws, wn, b)


def kernel(nfeat, efeat, timestamp, basis_freq, phase, Wt, bt,
           Ws1, bs1, Wn1, bn1, Ws2, bs2, Wn2, bn2, dst_ids, src_max_eid):
  e = efeat.shape[0]
  f = nfeat.shape[1]
  h = Wt.shape[1]
  ef = efeat.shape[1]

  gchunk = 128
  dst_i = dst_ids.astype(jnp.int32)
  src_i = src_max_eid.astype(jnp.int32)
  dst_idx2d = dst_i.reshape(e // gchunk, gchunk)
  src_idx2d = src_i.reshape(e // gchunk, gchunk)

  wt_u = Wt[:f]
  wt_e = Wt[f:f + ef]
  wt_t = Wt[f + ef:]

  proj = _tc_project(nfeat, wt_u)
  g0 = _sc_gather(proj, dst_idx2d, gchunk)

  ts2 = timestamp.reshape(e, 1)
  dst0 = _tc_encode(g0, efeat, ts2, wt_e, wt_t,
                    bt.reshape(1, h), basis_freq.reshape(1, h),
                    phase.reshape(1, h), block=2000)

  sblk = 256
  seg_col = dst_i.reshape(e, 1)
  seg_row = dst_i.reshape(e // sblk, 1, sblk)

  gsrc0 = _sc_gather(dst0, src_idx2d, gchunk)
  dst1 = _tc_sage_layer(dst0, gsrc0, seg_col, seg_row,
                        Ws1, Wn1, (bs1 + bn1).reshape(1, h), sblk)
  gsrc1 = _sc_gather(dst1, src_idx2d, gchunk)
  dst2 = _tc_sage_layer(dst1, gsrc1, seg_col, seg_row,
                        Ws2, Wn2, (bs2 + bn2).reshape(1, h), sblk)
  src = _sc_gather(dst2, src_idx2d, gchunk)
  return (src, dst2)


# speedup vs baseline: 1.8526x; 1.0866x over previous
import functools

import jax
import jax.numpy as jnp
from jax import lax
from jax.experimental import pallas as pl
from jax.experimental.pallas import tpu as pltpu
from jax.experimental.pallas import tpu_sc as plsc

_NC = 2
_NS = 16
_NW = _NC * _NS

_PREC = lax.Precision.DEFAULT


def _sc_gather(table, idx2d, chunk):
  nch = idx2d.shape[0]
  d = table.shape[1]
  nk = (nch + _NW - 1) // _NW
  mesh = plsc.VectorSubcoreMesh(
      core_axis_name="c", subcore_axis_name="s",
      num_cores=_NC, num_subcores=_NS)

  @functools.partial(
      pl.kernel,
      out_type=jax.ShapeDtypeStruct((nch * chunk, d), table.dtype),
      mesh=mesh,
      scratch_types=[
          pltpu.VMEM((chunk,), jnp.int32),
          pltpu.VMEM((chunk, d), table.dtype),
          pltpu.SemaphoreType.DMA,
      ],
  )
  def gather_kernel(table_hbm, idx_hbm, out_hbm, idx_v, rows_v, sem):
    w = lax.axis_index("s") * _NC + lax.axis_index("c")

    def body(k, _):
      cid = k * _NW + w

      @pl.when(cid < nch)
      def _():
        pltpu.sync_copy(idx_hbm.at[cid], idx_v)
        pltpu.async_copy(table_hbm.at[idx_v], rows_v, sem).wait()
        pltpu.sync_copy(rows_v, out_hbm.at[pl.ds(cid * chunk, chunk)])

      return 0

    lax.fori_loop(0, nk, body, 0, unroll=False)

  return gather_kernel(table, idx2d)


def _tc_project(nfeat, w):
  def body(nf_ref, w_ref, out_ref):
    out_ref[...] = jnp.dot(nf_ref[...], w_ref[...], precision=_PREC,
                           preferred_element_type=jnp.float32)

  return pl.pallas_call(
      body,
      out_shape=jax.ShapeDtypeStruct((nfeat.shape[0], w.shape[1]), jnp.float32),
  )(nfeat, w)


def _tc_encode(g0, efeat, ts, w_e, w_t, bt, bf, ph, block):
  e = g0.shape[0]
  h = w_t.shape[1]
  grid = e // block

  _c = (1.00000000e+00, -4.99999999e-01, 4.16666642e-02, -1.38888675e-03,
        2.48006914e-05, -2.75369917e-07, 2.06207486e-09, -9.77506520e-12)

  def _fast_cos(x):
    n = jnp.floor(x * jnp.float32(0.15915494309189535) + jnp.float32(0.5))
    r = (x - n * jnp.float32(6.28125)) - n * jnp.float32(0.0019353071795864769)
    s = r * r
    acc = jnp.float32(_c[7])
    for c in _c[6::-1]:
      acc = acc * s + jnp.float32(c)
    return acc

  def body(g_ref, ef_ref, ts_ref, we_ref, wt_ref, bt_ref, bf_ref, ph_ref, out_ref):
    te = _fast_cos(ts_ref[...] * bf_ref[...] + ph_ref[...])
    acc = g_ref[...] + bt_ref[...]
    acc += jnp.dot(ef_ref[...], we_ref[...], precision=_PREC,
                   preferred_element_type=jnp.float32)
    acc += jnp.dot(te, wt_ref[...], precision=_PREC,
                   preferred_element_type=jnp.float32)
    out_ref[...] = jnp.maximum(acc, 0.0)

  return pl.pallas_call(
      body,
      grid=(grid,),
      in_specs=[
          pl.BlockSpec((block, h), lambda i: (i, 0)),
          pl.BlockSpec((block, efeat.shape[1]), lambda i: (i, 0)),
          pl.BlockSpec((block, 1), lambda i: (i, 0)),
          pl.BlockSpec(w_e.shape, lambda i: (0, 0)),
          pl.BlockSpec(w_t.shape, lambda i: (0, 0)),
          pl.BlockSpec((1, h), lambda i: (0, 0)),
          pl.BlockSpec((1, h), lambda i: (0, 0)),
          pl.BlockSpec((1, h), lambda i: (0, 0)),
      ],
      out_specs=pl.BlockSpec((block, h), lambda i: (i, 0)),
      out_shape=jax.ShapeDtypeStruct((e, h), jnp.float32),
      compiler_params=pltpu.CompilerParams(
          dimension_semantics=("parallel",)),
  )(g0, efeat, ts, w_e, w_t, bt, bf, ph)


def _tc_sage_layer(dst, gsrc, seg_col, seg_row, ws, wn, b, block):
  e, h = dst.shape
  grid = e // block

  def body(dst_ref, gs_ref, sc_ref, sr_ref, ws_ref, wn_ref, b_ref, out_ref,
           carry_sum, carry_cnt, carry_seg):
    @pl.when(pl.program_id(0) == 0)
    def _():
      carry_seg[0] = -1
      carry_cnt[0] = 0.0
      carry_sum[...] = jnp.zeros_like(carry_sum)

    seg_c = sc_ref[...]
    seg_r = sr_ref[0]
    rowid = lax.broadcasted_iota(jnp.int32, (block, block), 0)
    colid = lax.broadcasted_iota(jnp.int32, (block, block), 1)
    mask = (colid <= rowid) & (seg_c == seg_r)
    mf = mask.astype(jnp.float32)
    cs = jnp.dot(mf, gs_ref[...], precision=_PREC,
                 preferred_element_type=jnp.float32)
    cntl = jnp.sum(mf, axis=1, keepdims=True)

    from_carry = (seg_c == carry_seg[0]).astype(jnp.float32)
    total = cs + from_carry * carry_sum[...]
    cnt = cntl + from_carry * carry_cnt[0]
    agg = total / cnt

    carry_sum[...] = total[block - 1:block, :]
    carry_cnt[0] = cnt[block - 1, 0]
    carry_seg[0] = seg_c[block - 1, 0]

    acc = jnp.dot(dst_ref[...], ws_ref[...], precision=_PREC,
                  preferred_element_type=jnp.float32)
    acc += jnp.dot(agg, wn_ref[...], precision=_PREC,
                   preferred_element_type=jnp.float32)
    out_ref[...] = jnp.maximum(acc + b_ref[...], 0.0)

  return pl.pallas_call(
      body,
      grid=(grid,),
      in_specs=[
          pl.BlockSpec((block, h), lambda i: (i, 0)),
          pl.BlockSpec((block, h), lambda i: (i, 0)),
          pl.BlockSpec((block, 1), lambda i: (i, 0)),
          pl.BlockSpec((1, 1, block), lambda i: (i, 0, 0)),
          pl.BlockSpec((h, h), lambda i: (0, 0)),
          pl.BlockSpec((h, h), lambda i: (0, 0)),
          pl.BlockSpec((1, h), lambda i: (0, 0)),
      ],
      out_specs=pl.BlockSpec((block, h), lambda i: (i, 0)),
      out_shape=jax.ShapeDtypeStruct((e, h), jnp.float32),
      scratch_shapes=[
          pltpu.VMEM((1, h), jnp.float32),
          pltpu.SMEM((1,), jnp.float32),
          pltpu.SMEM((1,), jnp.int32),
      ],
      compiler_params=pltpu.CompilerParams(
          dimension_semantics=("arbitrary",)),
  )(dst, gsrc, seg_col, seg_row, ws, wn, b)


def kernel(nfeat, efeat, timestamp, basis_freq, phase, Wt, bt,
           Ws1, bs1, Wn1, bn1, Ws2, bs2, Wn2, bn2, dst_ids, src_max_eid):
  e = efeat.shape[0]
  f = nfeat.shape[1]
  h = Wt.shape[1]
  ef = efeat.shape[1]

  gchunk = 128
  dst_i = dst_ids.astype(jnp.int32)
  src_i = src_max_eid.astype(jnp.int32)
  dst_idx2d = dst_i.reshape(e // gchunk, gchunk)
  src_idx2d = src_i.reshape(e // gchunk, gchunk)

  wt_u = Wt[:f]
  wt_e = Wt[f:f + ef]
  wt_t = Wt[f + ef:]

  proj = _tc_project(nfeat, wt_u)
  g0 = _sc_gather(proj, dst_idx2d, gchunk)

  ts2 = timestamp.reshape(e, 1)
  dst0 = _tc_encode(g0, efeat, ts2, wt_e, wt_t,
                    bt.reshape(1, h), basis_freq.reshape(1, h),
                    phase.reshape(1, h), block=2000)

  sblk = 256
  seg_col = dst_i.reshape(e, 1)
  seg_row = dst_i.reshape(e // sblk, 1, sblk)

  gsrc0 = _sc_gather(dst0, src_idx2d, gchunk)
  dst1 = _tc_sage_layer(dst0, gsrc0, seg_col, seg_row,
                        Ws1, Wn1, (bs1 + bn1).reshape(1, h), sblk)
  gsrc1 = _sc_gather(dst1, src_idx2d, gchunk)
  dst2 = _tc_sage_layer(dst1, gsrc1, seg_col, seg_row,
                        Ws2, Wn2, (bs2 + bn2).reshape(1, h), sblk)
  src = _sc_gather(dst2, src_idx2d, gchunk)
  return (src, dst2)


# speedup vs baseline: 1.9576x; 1.0567x over previous
import functools

import jax
import jax.numpy as jnp
from jax import lax
from jax.experimental import pallas as pl
from jax.experimental.pallas import tpu as pltpu
from jax.experimental.pallas import tpu_sc as plsc

_NC = 2
_NS = 16
_NW = _NC * _NS

_PREC = lax.Precision.DEFAULT


def _sc_gather(table, idx2d, chunk):
  nch = idx2d.shape[0]
  d = table.shape[1]
  nk = (nch + _NW - 1) // _NW
  mesh = plsc.VectorSubcoreMesh(
      core_axis_name="c", subcore_axis_name="s",
      num_cores=_NC, num_subcores=_NS)

  @functools.partial(
      pl.kernel,
      out_type=jax.ShapeDtypeStruct((nch * chunk, d), table.dtype),
      mesh=mesh,
      scratch_types=[
          pltpu.VMEM((chunk,), jnp.int32),
          pltpu.VMEM((chunk, d), table.dtype),
          pltpu.SemaphoreType.DMA,
      ],
  )
  def gather_kernel(table_hbm, idx_hbm, out_hbm, idx_v, rows_v, sem):
    w = lax.axis_index("s") * _NC + lax.axis_index("c")

    def body(k, _):
      cid = k * _NW + w

      @pl.when(cid < nch)
      def _():
        pltpu.sync_copy(idx_hbm.at[cid], idx_v)
        pltpu.async_copy(table_hbm.at[idx_v], rows_v, sem).wait()
        pltpu.sync_copy(rows_v, out_hbm.at[pl.ds(cid * chunk, chunk)])

      return 0

    lax.fori_loop(0, nk, body, 0, unroll=False)

  return gather_kernel(table, idx2d)


def _tc_project(nfeat, w):
  def body(nf_ref, w_ref, out_ref):
    out_ref[...] = jnp.dot(nf_ref[...], w_ref[...], precision=_PREC,
                           preferred_element_type=jnp.float32)

  return pl.pallas_call(
      body,
      out_shape=jax.ShapeDtypeStruct((nfeat.shape[0], w.shape[1]), jnp.float32),
  )(nfeat, w)


def _tc_encode(g0, efeat, ts, w_e, w_t, bt, bf, ph, block):
  e = g0.shape[0]
  h = w_t.shape[1]
  grid = e // block

  _c = (1.00000000e+00, -4.99999999e-01, 4.16666642e-02, -1.38888675e-03,
        2.48006914e-05, -2.75369917e-07, 2.06207486e-09, -9.77506520e-12)

  def _fast_cos(x):
    n = jnp.floor(x * jnp.float32(0.15915494309189535) + jnp.float32(0.5))
    r = (x - n * jnp.float32(6.28125)) - n * jnp.float32(0.0019353071795864769)
    s = r * r
    acc = jnp.float32(_c[7])
    for c in _c[6::-1]:
      acc = acc * s + jnp.float32(c)
    return acc

  def body(g_ref, ef_ref, ts_ref, we_ref, wt_ref, bt_ref, bf_ref, ph_ref, out_ref):
    t_col = ts_ref[0].reshape(block, 1)
    te = _fast_cos(t_col * bf_ref[...] + ph_ref[...])
    acc = g_ref[...] + bt_ref[...]
    acc += jnp.dot(ef_ref[...], we_ref[...], precision=_PREC,
                   preferred_element_type=jnp.float32)
    acc += jnp.dot(te, wt_ref[...], precision=_PREC,
                   preferred_element_type=jnp.float32)
    out_ref[...] = jnp.maximum(acc, 0.0)

  return pl.pallas_call(
      body,
      grid=(grid,),
      in_specs=[
          pl.BlockSpec((block, h), lambda i: (i, 0)),
          pl.BlockSpec((block, efeat.shape[1]), lambda i: (i, 0)),
          pl.BlockSpec((1, 1, block), lambda i: (i, 0, 0)),
          pl.BlockSpec(w_e.shape, lambda i: (0, 0)),
          pl.BlockSpec(w_t.shape, lambda i: (0, 0)),
          pl.BlockSpec((1, h), lambda i: (0, 0)),
          pl.BlockSpec((1, h), lambda i: (0, 0)),
          pl.BlockSpec((1, h), lambda i: (0, 0)),
      ],
      out_specs=pl.BlockSpec((block, h), lambda i: (i, 0)),
      out_shape=jax.ShapeDtypeStruct((e, h), jnp.float32),
      compiler_params=pltpu.CompilerParams(
          dimension_semantics=("parallel",)),
  )(g0, efeat, ts, w_e, w_t, bt, bf, ph)


def _tc_sage_layer(dst, gsrc, seg_row, ws, wn, b, block):
  e, h = dst.shape
  grid = e // block

  def body(dst_ref, gs_ref, sr_ref, ws_ref, wn_ref, b_ref, out_ref,
           carry_sum, carry_cnt, carry_seg):
    @pl.when(pl.program_id(0) == 0)
    def _():
      carry_seg[0] = -1
      carry_cnt[0] = 0.0
      carry_sum[...] = jnp.zeros_like(carry_sum)

    seg_r = sr_ref[0]
    seg_c = seg_r.reshape(block, 1)
    rowid = lax.broadcasted_iota(jnp.int32, (block, block), 0)
    colid = lax.broadcasted_iota(jnp.int32, (block, block), 1)
    mask = (colid <= rowid) & (seg_c == seg_r)
    mf = mask.astype(jnp.float32)
    cs = jnp.dot(mf, gs_ref[...], precision=_PREC,
                 preferred_element_type=jnp.float32)
    cntl = jnp.sum(mf, axis=1, keepdims=True)

    from_carry = (seg_c == carry_seg[0]).astype(jnp.float32)
    total = cs + from_carry * carry_sum[...]
    cnt = cntl + from_carry * carry_cnt[0]
    agg = total / cnt

    carry_sum[...] = total[block - 1:block, :]
    carry_cnt[0] = cnt[block - 1, 0]
    carry_seg[0] = seg_c[block - 1, 0]

    acc = jnp.dot(dst_ref[...], ws_ref[...], precision=_PREC,
                  preferred_element_type=jnp.float32)
    acc += jnp.dot(agg, wn_ref[...], precision=_PREC,
                   preferred_element_type=jnp.float32)
    out_ref[...] = jnp.maximum(acc + b_ref[...], 0.0)

  return pl.pallas_call(
      body,
      grid=(grid,),
      in_specs=[
          pl.BlockSpec((block, h), lambda i: (i, 0)),
          pl.BlockSpec((block, h), lambda i: (i, 0)),
          pl.BlockSpec((1, 1, block), lambda i: (i, 0, 0)),
          pl.BlockSpec((h, h), lambda i: (0, 0)),
          pl.BlockSpec((h, h), lambda i: (0, 0)),
          pl.BlockSpec((1, h), lambda i: (0, 0)),
      ],
      out_specs=pl.BlockSpec((block, h), lambda i: (i, 0)),
      out_shape=jax.ShapeDtypeStruct((e, h), jnp.float32),
      scratch_shapes=[
          pltpu.VMEM((1, h), jnp.float32),
          pltpu.SMEM((1,), jnp.float32),
          pltpu.SMEM((1,), jnp.int32),
      ],
      compiler_params=pltpu.CompilerParams(
          dimension_semantics=("arbitrary",)),
  )(dst, gsrc, seg_row, ws, wn, b)


def kernel(nfeat, efeat, timestamp, basis_freq, phase, Wt, bt,
           Ws1, bs1, Wn1, bn1, Ws2, bs2, Wn2, bn2, dst_ids, src_max_eid):
  e = efeat.shape[0]
  f = nfeat.shape[1]
  h = Wt.shape[1]
  ef = efeat.shape[1]

  gchunk = 128
  dst_i = dst_ids.astype(jnp.int32)
  src_i = src_max_eid.astype(jnp.int32)
  dst_idx2d = dst_i.reshape(e // gchunk, gchunk)
  src_idx2d = src_i.reshape(e // gchunk, gchunk)

  wt_u = Wt[:f]
  wt_e = Wt[f:f + ef]
  wt_t = Wt[f + ef:]

  proj = _tc_project(nfeat, wt_u)
  g0 = _sc_gather(proj, dst_idx2d, gchunk)

  eblk = 2000
  ts3 = timestamp.reshape(e // eblk, 1, eblk)
  dst0 = _tc_encode(g0, efeat, ts3, wt_e, wt_t,
                    bt.reshape(1, h), basis_freq.reshape(1, h),
                    phase.reshape(1, h), block=eblk)

  sblk = 256
  seg_row = dst_i.reshape(e // sblk, 1, sblk)

  gsrc0 = _sc_gather(dst0, src_idx2d, gchunk)
  dst1 = _tc_sage_layer(dst0, gsrc0, seg_row,
                        Ws1, Wn1, (bs1 + bn1).reshape(1, h), sblk)
  gsrc1 = _sc_gather(dst1, src_idx2d, gchunk)
  dst2 = _tc_sage_layer(dst1, gsrc1, seg_row,
                        Ws2, Wn2, (bs2 + bn2).reshape(1, h), sblk)
  src = _sc_gather(dst2, src_idx2d, gchunk)
  return (src, dst2)


# speedup vs baseline: 2.6133x; 1.3350x over previous
import functools

import jax
import jax.numpy as jnp
from jax import lax
from jax.experimental import pallas as pl
from jax.experimental.pallas import tpu as pltpu
from jax.experimental.pallas import tpu_sc as plsc

_NC = 2
_NS = 16
_NW = _NC * _NS

_PREC = lax.Precision.DEFAULT


def _sc_gather(table, idx2d, chunk):
  nch = idx2d.shape[0]
  d = table.shape[1]
  nk = (nch + _NW - 1) // _NW
  mesh = plsc.VectorSubcoreMesh(
      core_axis_name="c", subcore_axis_name="s",
      num_cores=_NC, num_subcores=_NS)

  @functools.partial(
      pl.kernel,
      out_type=jax.ShapeDtypeStruct((nch * chunk, d), table.dtype),
      mesh=mesh,
      scratch_types=[
          pltpu.VMEM((chunk,), jnp.int32),
          pltpu.VMEM((chunk, d), table.dtype),
          pltpu.SemaphoreType.DMA,
      ],
  )
  def gather_kernel(table_hbm, idx_hbm, out_hbm, idx_v, rows_v, sem):
    w = lax.axis_index("s") * _NC + lax.axis_index("c")

    def body(k, _):
      cid = k * _NW + w

      @pl.when(cid < nch)
      def _():
        pltpu.sync_copy(idx_hbm.at[cid], idx_v)
        pltpu.async_copy(table_hbm.at[idx_v], rows_v, sem).wait()
        pltpu.sync_copy(rows_v, out_hbm.at[pl.ds(cid * chunk, chunk)])

      return 0

    lax.fori_loop(0, nk, body, 0, unroll=False)

  return gather_kernel(table, idx2d)


def _tc_project(nfeat, w):
  def body(nf_ref, w_ref, out_ref):
    out_ref[...] = jnp.dot(nf_ref[...], w_ref[...], precision=_PREC,
                           preferred_element_type=jnp.float32)

  return pl.pallas_call(
      body,
      out_shape=jax.ShapeDtypeStruct((nfeat.shape[0], w.shape[1]), jnp.float32),
  )(nfeat, w)


def _tc_encode(g0, efeat, ts, w_e, w_t, bt, bf, ph, block):
  e = g0.shape[0]
  h = w_t.shape[1]
  grid = e // block

  _c = (1.00000000e+00, -4.99999999e-01, 4.16666642e-02, -1.38888675e-03,
        2.48006914e-05, -2.75369917e-07, 2.06207486e-09, -9.77506520e-12)

  def _fast_cos(x):
    n = jnp.floor(x * jnp.float32(0.15915494309189535) + jnp.float32(0.5))
    r = (x - n * jnp.float32(6.28125)) - n * jnp.float32(0.0019353071795864769)
    s = r * r
    acc = jnp.float32(_c[7])
    for c in _c[6::-1]:
      acc = acc * s + jnp.float32(c)
    return acc

  def body(g_ref, ef_ref, ts_ref, we_ref, wt_ref, bt_ref, bf_ref, ph_ref, out_ref):
    t_col = ts_ref[0].reshape(block, 1)
    te = _fast_cos(t_col * bf_ref[...] + ph_ref[...])
    acc = g_ref[...] + bt_ref[...]
    acc += lax.dot_general(ef_ref[...], we_ref[...],
                           dimension_numbers=(((0,), (0,)), ((), ())),
                           precision=_PREC,
                           preferred_element_type=jnp.float32)
    acc += jnp.dot(te, wt_ref[...], precision=_PREC,
                   preferred_element_type=jnp.float32)
    out_ref[...] = jnp.maximum(acc, 0.0)

  return pl.pallas_call(
      body,
      grid=(grid,),
      in_specs=[
          pl.BlockSpec((block, h), lambda i: (i, 0)),
          pl.BlockSpec((efeat.shape[0], block), lambda i: (0, i)),
          pl.BlockSpec((1, 1, block), lambda i: (i, 0, 0)),
          pl.BlockSpec(w_e.shape, lambda i: (0, 0)),
          pl.BlockSpec(w_t.shape, lambda i: (0, 0)),
          pl.BlockSpec((1, h), lambda i: (0, 0)),
          pl.BlockSpec((1, h), lambda i: (0, 0)),
          pl.BlockSpec((1, h), lambda i: (0, 0)),
      ],
      out_specs=pl.BlockSpec((block, h), lambda i: (i, 0)),
      out_shape=jax.ShapeDtypeStruct((e, h), jnp.float32),
      compiler_params=pltpu.CompilerParams(
          dimension_semantics=("parallel",)),
  )(g0, efeat, ts, w_e, w_t, bt, bf, ph)


def _tc_sage_layer(dst, gsrc, seg_row, ws, wn, b, block):
  e, h = dst.shape
  grid = e // block

  def body(dst_ref, gs_ref, sr_ref, ws_ref, wn_ref, b_ref, out_ref,
           carry_sum, carry_cnt, carry_seg):
    @pl.when(pl.program_id(0) == 0)
    def _():
      carry_seg[0] = -1
      carry_cnt[0] = 0.0
      carry_sum[...] = jnp.zeros_like(carry_sum)

    seg_r = sr_ref[0]
    seg_c = seg_r.reshape(block, 1)
    rowid = lax.broadcasted_iota(jnp.int32, (block, block), 0)
    colid = lax.broadcasted_iota(jnp.int32, (block, block), 1)
    mask = (colid <= rowid) & (seg_c == seg_r)
    mf = mask.astype(jnp.float32)
    cs = jnp.dot(mf, gs_ref[...], precision=_PREC,
                 preferred_element_type=jnp.float32)
    cntl = jnp.sum(mf, axis=1, keepdims=True)

    from_carry = (seg_c == carry_seg[0]).astype(jnp.float32)
    total = cs + from_carry * carry_sum[...]
    cnt = cntl + from_carry * carry_cnt[0]
    agg = total / cnt

    carry_sum[...] = total[block - 1:block, :]
    carry_cnt[0] = cnt[block - 1, 0]
    carry_seg[0] = seg_c[block - 1, 0]

    acc = jnp.dot(dst_ref[...], ws_ref[...], precision=_PREC,
                  preferred_element_type=jnp.float32)
    acc += jnp.dot(agg, wn_ref[...], precision=_PREC,
                   preferred_element_type=jnp.float32)
    out_ref[...] = jnp.maximum(acc + b_ref[...], 0.0)

  return pl.pallas_call(
      body,
      grid=(grid,),
      in_specs=[
          pl.BlockSpec((block, h), lambda i: (i, 0)),
          pl.BlockSpec((block, h), lambda i: (i, 0)),
          pl.BlockSpec((1, 1, block), lambda i: (i, 0, 0)),
          pl.BlockSpec((h, h), lambda i: (0, 0)),
          pl.BlockSpec((h, h), lambda i: (0, 0)),
          pl.BlockSpec((1, h), lambda i: (0, 0)),
      ],
      out_specs=pl.BlockSpec((block, h), lambda i: (i, 0)),
      out_shape=jax.ShapeDtypeStruct((e, h), jnp.float32),
      scratch_shapes=[
          pltpu.VMEM((1, h), jnp.float32),
          pltpu.SMEM((1,), jnp.float32),
          pltpu.SMEM((1,), jnp.int32),
      ],
      compiler_params=pltpu.CompilerParams(
          dimension_semantics=("arbitrary",)),
  )(dst, gsrc, seg_row, ws, wn, b)


def kernel(nfeat, efeat, timestamp, basis_freq, phase, Wt, bt,
           Ws1, bs1, Wn1, bn1, Ws2, bs2, Wn2, bn2, dst_ids, src_max_eid):
  e = efeat.shape[0]
  f = nfeat.shape[1]
  h = Wt.shape[1]
  ef = efeat.shape[1]

  gchunk = 128
  dst_i = dst_ids.astype(jnp.int32)
  src_i = src_max_eid.astype(jnp.int32)
  dst_idx2d = dst_i.reshape(e // gchunk, gchunk)
  src_idx2d = src_i.reshape(e // gchunk, gchunk)

  wt_u = Wt[:f]
  wt_e = Wt[f:f + ef]
  wt_t = Wt[f + ef:]

  proj = _tc_project(nfeat, wt_u)
  g0 = _sc_gather(proj, dst_idx2d, gchunk)

  eblk = 1280
  ts3 = timestamp.reshape(e // eblk, 1, eblk)
  ef_t = jnp.swapaxes(efeat, 0, 1)
  dst0 = _tc_encode(g0, ef_t, ts3, wt_e, wt_t,
                    bt.reshape(1, h), basis_freq.reshape(1, h),
                    phase.reshape(1, h), block=eblk)

  sblk = 640
  seg_row = dst_i.reshape(e // sblk, 1, sblk)

  gsrc0 = _sc_gather(dst0, src_idx2d, gchunk)
  dst1 = _tc_sage_layer(dst0, gsrc0, seg_row,
                        Ws1, Wn1, (bs1 + bn1).reshape(1, h), sblk)
  gsrc1 = _sc_gather(dst1, src_idx2d, gchunk)
  dst2 = _tc_sage_layer(dst1, gsrc1, seg_row,
                        Ws2, Wn2, (bs2 + bn2).reshape(1, h), sblk)
  src = _sc_gather(dst2, src_idx2d, gchunk)
  return (src, dst2)


# speedup vs baseline: 2.9114x; 1.1141x over previous
import functools

import jax
import jax.numpy as jnp
from jax import lax
from jax.experimental import pallas as pl
from jax.experimental.pallas import tpu as pltpu
from jax.experimental.pallas import tpu_sc as plsc

_NC = 2
_NS = 16
_NW = _NC * _NS

_PREC = lax.Precision.DEFAULT


def _sc_gather(table, idx3w, nch, chunk):
  nk = idx3w.shape[1]
  d = table.shape[1]
  mesh = plsc.VectorSubcoreMesh(
      core_axis_name="c", subcore_axis_name="s",
      num_cores=_NC, num_subcores=_NS)

  @functools.partial(
      pl.kernel,
      out_type=jax.ShapeDtypeStruct((nch * chunk, d), table.dtype),
      mesh=mesh,
      scratch_types=[
          pltpu.VMEM((nk, chunk), jnp.int32),
          pltpu.VMEM((chunk, d), table.dtype),
          pltpu.VMEM((chunk, d), table.dtype),
          pltpu.SemaphoreType.DMA,
          pltpu.SemaphoreType.DMA,
          pltpu.SemaphoreType.DMA,
      ],
  )
  def gather_kernel(table_hbm, idx_hbm, out_hbm, idx_v, r0, r1, sg, so0, so1):
    w = lax.axis_index("s") * _NC + lax.axis_index("c")
    pltpu.sync_copy(idx_hbm.at[w], idx_v)
    vc = (nch - w + _NW - 1) // _NW

    def half(k, rb, so):
      cid = k * _NW + w

      @pl.when(k < vc)
      def _():
        @pl.when(k >= 2)
        def _():
          pltpu.make_async_copy(rb, out_hbm.at[pl.ds(0, chunk)], so).wait()

        pltpu.async_copy(table_hbm.at[idx_v.at[k]], rb, sg).wait()
        pltpu.async_copy(rb, out_hbm.at[pl.ds(cid * chunk, chunk)], so)

    def body(j, _):
      half(2 * j, r0, so0)
      half(2 * j + 1, r1, so1)
      return 0

    lax.fori_loop(0, (nk + 1) // 2, body, 0, unroll=False)

    for p, so in ((0, so0), (1, so1)):
      @pl.when((vc + 1 - p) // 2 >= 1)
      def _():
        pltpu.make_async_copy(r0 if p == 0 else r1,
                              out_hbm.at[pl.ds(0, chunk)], so).wait()

  return gather_kernel(table, idx3w)


def _tc_project(nfeat, w):
  def body(nf_ref, w_ref, out_ref):
    out_ref[...] = jnp.dot(nf_ref[...], w_ref[...], precision=_PREC,
                           preferred_element_type=jnp.float32)

  return pl.pallas_call(
      body,
      out_shape=jax.ShapeDtypeStruct((nfeat.shape[0], w.shape[1]), jnp.float32),
  )(nfeat, w)


def _tc_encode(g0, efeat, ts, w_e, w_t, bt, bf, ph, block):
  e = g0.shape[0]
  h = w_t.shape[1]
  grid = e // block

  _c = (1.00000000e+00, -4.99999999e-01, 4.16666642e-02, -1.38888675e-03,
        2.48006914e-05, -2.75369917e-07, 2.06207486e-09, -9.77506520e-12)

  def _fast_cos(x):
    n = jnp.floor(x * jnp.float32(0.15915494309189535) + jnp.float32(0.5))
    r = (x - n * jnp.float32(6.28125)) - n * jnp.float32(0.0019353071795864769)
    s = r * r
    acc = jnp.float32(_c[7])
    for c in _c[6::-1]:
      acc = acc * s + jnp.float32(c)
    return acc

  def body(g_ref, ef_ref, ts_ref, we_ref, wt_ref, bt_ref, bf_ref, ph_ref, out_ref):
    t_col = ts_ref[0].reshape(block, 1)
    te = _fast_cos(t_col * bf_ref[...] + ph_ref[...])
    acc = g_ref[...] + bt_ref[...]
    acc += lax.dot_general(ef_ref[...], we_ref[...],
                           dimension_numbers=(((0,), (0,)), ((), ())),
                           precision=_PREC,
                           preferred_element_type=jnp.float32)
    acc += jnp.dot(te, wt_ref[...], precision=_PREC,
                   preferred_element_type=jnp.float32)
    out_ref[...] = jnp.maximum(acc, 0.0)

  return pl.pallas_call(
      body,
      grid=(grid,),
      in_specs=[
          pl.BlockSpec((block, h), lambda i: (i, 0)),
          pl.BlockSpec((efeat.shape[0], block), lambda i: (0, i)),
          pl.BlockSpec((1, 1, block), lambda i: (i, 0, 0)),
          pl.BlockSpec(w_e.shape, lambda i: (0, 0)),
          pl.BlockSpec(w_t.shape, lambda i: (0, 0)),
          pl.BlockSpec((1, h), lambda i: (0, 0)),
          pl.BlockSpec((1, h), lambda i: (0, 0)),
          pl.BlockSpec((1, h), lambda i: (0, 0)),
      ],
      out_specs=pl.BlockSpec((block, h), lambda i: (i, 0)),
      out_shape=jax.ShapeDtypeStruct((e, h), jnp.float32),
      compiler_params=pltpu.CompilerParams(
          dimension_semantics=("parallel",)),
  )(g0, efeat, ts, w_e, w_t, bt, bf, ph)


def _tc_sage_layer(dst, gsrc, seg_row, ws, wn, b, block):
  e, h = dst.shape
  grid = e // block

  def body(dst_ref, gs_ref, sr_ref, ws_ref, wn_ref, b_ref, out_ref,
           carry_sum, carry_cnt, carry_seg):
    @pl.when(pl.program_id(0) == 0)
    def _():
      carry_seg[0] = -1
      carry_cnt[0] = 0.0
      carry_sum[...] = jnp.zeros_like(carry_sum)

    seg_r = sr_ref[0]
    seg_c = seg_r.reshape(block, 1)
    rowid = lax.broadcasted_iota(jnp.int32, (block, block), 0)
    colid = lax.broadcasted_iota(jnp.int32, (block, block), 1)
    mask = (colid <= rowid) & (seg_c == seg_r)
    mf = mask.astype(jnp.float32)
    cs = jnp.dot(mf, gs_ref[...], precision=_PREC,
                 preferred_element_type=jnp.float32)
    cntl = jnp.sum(mf, axis=1, keepdims=True)

    from_carry = (seg_c == carry_seg[0]).astype(jnp.float32)
    total = cs + from_carry * carry_sum[...]
    cnt = cntl + from_carry * carry_cnt[0]
    agg = total / cnt

    carry_sum[...] = total[block - 1:block, :]
    carry_cnt[0] = cnt[block - 1, 0]
    carry_seg[0] = seg_c[block - 1, 0]

    acc = jnp.dot(dst_ref[...], ws_ref[...], precision=_PREC,
                  preferred_element_type=jnp.float32)
    acc += jnp.dot(agg, wn_ref[...], precision=_PREC,
                   preferred_element_type=jnp.float32)
    out_ref[...] = jnp.maximum(acc + b_ref[...], 0.0)

  return pl.pallas_call(
      body,
      grid=(grid,),
      in_specs=[
          pl.BlockSpec((block, h), lambda i: (i, 0)),
          pl.BlockSpec((block, h), lambda i: (i, 0)),
          pl.BlockSpec((1, 1, block), lambda i: (i, 0, 0)),
          pl.BlockSpec((h, h), lambda i: (0, 0)),
          pl.BlockSpec((h, h), lambda i: (0, 0)),
          pl.BlockSpec((1, h), lambda i: (0, 0)),
      ],
      out_specs=pl.BlockSpec((block, h), lambda i: (i, 0)),
      out_shape=jax.ShapeDtypeStruct((e, h), jnp.float32),
      scratch_shapes=[
          pltpu.VMEM((1, h), jnp.float32),
          pltpu.SMEM((1,), jnp.float32),
          pltpu.SMEM((1,), jnp.int32),
      ],
      compiler_params=pltpu.CompilerParams(
          dimension_semantics=("arbitrary",)),
  )(dst, gsrc, seg_row, ws, wn, b)


def kernel(nfeat, efeat, timestamp, basis_freq, phase, Wt, bt,
           Ws1, bs1, Wn1, bn1, Ws2, bs2, Wn2, bn2, dst_ids, src_max_eid):
  e = efeat.shape[0]
  f = nfeat.shape[1]
  h = Wt.shape[1]
  ef = efeat.shape[1]

  gchunk = 128
  dst_i = dst_ids.astype(jnp.int32)
  src_i = src_max_eid.astype(jnp.int32)
  nch = e // gchunk
  nk = (nch + _NW - 1) // _NW
  pad = nk * _NW - nch

  def _worker_major(idx):
    idx2d = idx.reshape(nch, gchunk)
    if pad:
      idx2d = jnp.concatenate(
          [idx2d, jnp.zeros((pad, gchunk), jnp.int32)], axis=0)
    return idx2d.reshape(nk, _NW, gchunk).swapaxes(0, 1)

  dst_idx3 = _worker_major(dst_i)
  src_idx3 = _worker_major(src_i)

  wt_u = Wt[:f]
  wt_e = Wt[f:f + ef]
  wt_t = Wt[f + ef:]

  proj = _tc_project(nfeat, wt_u)
  g0 = _sc_gather(proj, dst_idx3, nch, gchunk)

  eblk = 1280
  ts3 = timestamp.reshape(e // eblk, 1, eblk)
  ef_t = jnp.swapaxes(efeat, 0, 1)
  dst0 = _tc_encode(g0, ef_t, ts3, wt_e, wt_t,
                    bt.reshape(1, h), basis_freq.reshape(1, h),
                    phase.reshape(1, h), block=eblk)

  sblk = 640
  seg_row = dst_i.reshape(e // sblk, 1, sblk)

  gsrc0 = _sc_gather(dst0, src_idx3, nch, gchunk)
  dst1 = _tc_sage_layer(dst0, gsrc0, seg_row,
                        Ws1, Wn1, (bs1 + bn1).reshape(1, h), sblk)
  gsrc1 = _sc_gather(dst1, src_idx3, nch, gchunk)
  dst2 = _tc_sage_layer(dst1, gsrc1, seg_row,
                        Ws2, Wn2, (bs2 + bn2).reshape(1, h), sblk)
  src = _sc_gather(dst2, src_idx3, nch, gchunk)
  return (src, dst2)


# speedup vs baseline: 3.4034x; 1.1690x over previous
import functools

import jax
import jax.numpy as jnp
from jax import lax
from jax.experimental import pallas as pl
from jax.experimental.pallas import tpu as pltpu
from jax.experimental.pallas import tpu_sc as plsc

_NC = 2
_NS = 16
_NW = _NC * _NS

_PREC = lax.Precision.DEFAULT


def _sc_gather(table, idx3w, nch, chunk):
  nk = idx3w.shape[1]
  d = table.shape[1]
  mesh = plsc.VectorSubcoreMesh(
      core_axis_name="c", subcore_axis_name="s",
      num_cores=_NC, num_subcores=_NS)

  @functools.partial(
      pl.kernel,
      out_type=jax.ShapeDtypeStruct((nch * chunk, d), table.dtype),
      mesh=mesh,
      scratch_types=[
          pltpu.VMEM((nk, chunk), jnp.int32),
          pltpu.VMEM((chunk, d), table.dtype),
          pltpu.VMEM((chunk, d), table.dtype),
          pltpu.SemaphoreType.DMA,
          pltpu.SemaphoreType.DMA,
          pltpu.SemaphoreType.DMA,
      ],
  )
  def gather_kernel(table_hbm, idx_hbm, out_hbm, idx_v, r0, r1, sg, so0, so1):
    w = lax.axis_index("s") * _NC + lax.axis_index("c")
    pltpu.sync_copy(idx_hbm.at[w], idx_v)
    vc = (nch - w + _NW - 1) // _NW

    def half(k, rb, so):
      cid = k * _NW + w

      @pl.when(k < vc)
      def _():
        @pl.when(k >= 2)
        def _():
          pltpu.make_async_copy(rb, out_hbm.at[pl.ds(0, chunk)], so).wait()

        pltpu.async_copy(table_hbm.at[idx_v.at[k]], rb, sg).wait()
        pltpu.async_copy(rb, out_hbm.at[pl.ds(cid * chunk, chunk)], so)

    def body(j, _):
      half(2 * j, r0, so0)
      half(2 * j + 1, r1, so1)
      return 0

    lax.fori_loop(0, (nk + 1) // 2, body, 0, unroll=False)

    for p, so in ((0, so0), (1, so1)):
      @pl.when((vc + 1 - p) // 2 >= 1)
      def _():
        pltpu.make_async_copy(r0 if p == 0 else r1,
                              out_hbm.at[pl.ds(0, chunk)], so).wait()

  return gather_kernel(table, idx3w)


def _tc_project(nfeat, w):
  def body(nf_ref, w_ref, out_ref):
    out_ref[...] = jnp.dot(nf_ref[...], w_ref[...], precision=_PREC,
                           preferred_element_type=jnp.float32)

  return pl.pallas_call(
      body,
      out_shape=jax.ShapeDtypeStruct((nfeat.shape[0], w.shape[1]), jnp.float32),
  )(nfeat, w)


def _tc_encode(g0, efeat, ts, w_e, w_t, bt, bf, ph, block):
  e = g0.shape[0]
  h = w_t.shape[1]
  grid = e // block

  _c = (1.00000000e+00, -4.99999999e-01, 4.16666642e-02, -1.38888675e-03,
        2.48006914e-05, -2.75369917e-07, 2.06207486e-09, -9.77506520e-12)

  def _fast_cos(x):
    n = jnp.floor(x * jnp.float32(0.15915494309189535) + jnp.float32(0.5))
    r = (x - n * jnp.float32(6.28125)) - n * jnp.float32(0.0019353071795864769)
    s = r * r
    acc = jnp.float32(_c[7])
    for c in _c[6::-1]:
      acc = acc * s + jnp.float32(c)
    return acc

  def body(g_ref, ef_ref, ts_ref, we_ref, wt_ref, bt_ref, bf_ref, ph_ref, out_ref):
    t_col = ts_ref[0].reshape(block, 1)
    te = _fast_cos(t_col * bf_ref[...] + ph_ref[...])
    acc = g_ref[...] + bt_ref[...]
    acc += lax.dot_general(ef_ref[...], we_ref[...],
                           dimension_numbers=(((0,), (0,)), ((), ())),
                           precision=_PREC,
                           preferred_element_type=jnp.float32)
    acc += jnp.dot(te, wt_ref[...], precision=_PREC,
                   preferred_element_type=jnp.float32)
    out_ref[...] = jnp.maximum(acc, 0.0)

  return pl.pallas_call(
      body,
      grid=(grid,),
      in_specs=[
          pl.BlockSpec((block, h), lambda i: (i, 0)),
          pl.BlockSpec((efeat.shape[0], block), lambda i: (0, i)),
          pl.BlockSpec((1, 1, block), lambda i: (i, 0, 0)),
          pl.BlockSpec(w_e.shape, lambda i: (0, 0)),
          pl.BlockSpec(w_t.shape, lambda i: (0, 0)),
          pl.BlockSpec((1, h), lambda i: (0, 0)),
          pl.BlockSpec((1, h), lambda i: (0, 0)),
          pl.BlockSpec((1, h), lambda i: (0, 0)),
      ],
      out_specs=pl.BlockSpec((block, h), lambda i: (i, 0)),
      out_shape=jax.ShapeDtypeStruct((e, h), jnp.float32),
      compiler_params=pltpu.CompilerParams(
          dimension_semantics=("parallel",)),
  )(g0, efeat, ts, w_e, w_t, bt, bf, ph)


def _tc_sage_layer(dst, gsrc, seg_row, ws, wn, b, block):
  e, h = dst.shape
  grid = e // block

  sub = 320
  nsub = block // sub

  def body(dst_ref, gs_ref, sr_ref, ws_ref, wn_ref, b_ref, out_ref,
           carry_sum, carry_cnt, carry_seg):
    @pl.when(pl.program_id(0) == 0)
    def _():
      carry_seg[0] = -1
      carry_cnt[0] = 0.0
      carry_sum[...] = jnp.zeros_like(carry_sum)

    seg_full = sr_ref[0]
    rowid = lax.broadcasted_iota(jnp.int32, (sub, sub), 0)
    colid = lax.broadcasted_iota(jnp.int32, (sub, sub), 1)
    tri = colid <= rowid

    c_sum = carry_sum[...]
    c_cnt = carry_cnt[0]
    c_seg = carry_seg[0]
    aggs = []
    for p in range(nsub):
      seg_r = seg_full[:, p * sub:(p + 1) * sub]
      seg_c = seg_r.reshape(sub, 1)
      mf = (tri & (seg_c == seg_r)).astype(jnp.float32)
      cs = jnp.dot(mf, gs_ref[p * sub:(p + 1) * sub, :], precision=_PREC,
                   preferred_element_type=jnp.float32)
      cntl = jnp.sum(mf, axis=1, keepdims=True)
      fc = (seg_c == c_seg).astype(jnp.float32)
      total = cs + fc * c_sum
      cnt = cntl + fc * c_cnt
      aggs.append(total / cnt)
      c_sum = total[sub - 1:sub, :]
      c_cnt = cnt[sub - 1, 0]
      c_seg = seg_c[sub - 1, 0]

    agg = jnp.concatenate(aggs, axis=0)
    carry_sum[...] = c_sum
    carry_cnt[0] = c_cnt
    carry_seg[0] = c_seg

    acc = jnp.dot(dst_ref[...], ws_ref[...], precision=_PREC,
                  preferred_element_type=jnp.float32)
    acc += jnp.dot(agg, wn_ref[...], precision=_PREC,
                   preferred_element_type=jnp.float32)
    out_ref[...] = jnp.maximum(acc + b_ref[...], 0.0)

  return pl.pallas_call(
      body,
      grid=(grid,),
      in_specs=[
          pl.BlockSpec((block, h), lambda i: (i, 0)),
          pl.BlockSpec((block, h), lambda i: (i, 0)),
          pl.BlockSpec((1, 1, block), lambda i: (i, 0, 0)),
          pl.BlockSpec((h, h), lambda i: (0, 0)),
          pl.BlockSpec((h, h), lambda i: (0, 0)),
          pl.BlockSpec((1, h), lambda i: (0, 0)),
      ],
      out_specs=pl.BlockSpec((block, h), lambda i: (i, 0)),
      out_shape=jax.ShapeDtypeStruct((e, h), jnp.float32),
      scratch_shapes=[
          pltpu.VMEM((1, h), jnp.float32),
          pltpu.SMEM((1,), jnp.float32),
          pltpu.SMEM((1,), jnp.int32),
      ],
      compiler_params=pltpu.CompilerParams(
          dimension_semantics=("arbitrary",)),
  )(dst, gsrc, seg_row, ws, wn, b)


def kernel(nfeat, efeat, timestamp, basis_freq, phase, Wt, bt,
           Ws1, bs1, Wn1, bn1, Ws2, bs2, Wn2, bn2, dst_ids, src_max_eid):
  e = efeat.shape[0]
  f = nfeat.shape[1]
  h = Wt.shape[1]
  ef = efeat.shape[1]

  gchunk = 128
  dst_i = dst_ids.astype(jnp.int32)
  src_i = src_max_eid.astype(jnp.int32)
  nch = e // gchunk
  nk = (nch + _NW - 1) // _NW
  pad = nk * _NW - nch

  def _worker_major(idx):
    idx2d = idx.reshape(nch, gchunk)
    if pad:
      idx2d = jnp.concatenate(
          [idx2d, jnp.zeros((pad, gchunk), jnp.int32)], axis=0)
    return idx2d.reshape(nk, _NW, gchunk).swapaxes(0, 1)

  dst_idx3 = _worker_major(dst_i)
  src_idx3 = _worker_major(src_i)

  wt_u = Wt[:f]
  wt_e = Wt[f:f + ef]
  wt_t = Wt[f + ef:]

  proj = _tc_project(nfeat, wt_u)
  g0 = _sc_gather(proj, dst_idx3, nch, gchunk)

  eblk = 3200
  ts3 = timestamp.reshape(e // eblk, 1, eblk)
  ef_t = jnp.swapaxes(efeat, 0, 1)
  dst0 = _tc_encode(g0, ef_t, ts3, wt_e, wt_t,
                    bt.reshape(1, h), basis_freq.reshape(1, h),
                    phase.reshape(1, h), block=eblk)

  sblk = 1280
  seg_row = dst_i.reshape(e // sblk, 1, sblk)

  gsrc0 = _sc_gather(dst0, src_idx3, nch, gchunk)
  dst1 = _tc_sage_layer(dst0, gsrc0, seg_row,
                        Ws1, Wn1, (bs1 + bn1).reshape(1, h), sblk)
  gsrc1 = _sc_gather(dst1, src_idx3, nch, gchunk)
  dst2 = _tc_sage_layer(dst1, gsrc1, seg_row,
                        Ws2, Wn2, (bs2 + bn2).reshape(1, h), sblk)
  src = _sc_gather(dst2, src_idx3, nch, gchunk)
  return (src, dst2)


# speedup vs baseline: 3.9366x; 1.1567x over previous
import functools

import jax
import jax.numpy as jnp
from jax import lax
from jax.experimental import pallas as pl
from jax.experimental.pallas import tpu as pltpu
from jax.experimental.pallas import tpu_sc as plsc

_NC = 2
_NS = 16
_NW = _NC * _NS

_PREC = lax.Precision.DEFAULT


def _sc_gather(table, idx3w, nch, chunk, contig=False):
  nk = idx3w.shape[1]
  d = table.shape[1]
  mesh = plsc.VectorSubcoreMesh(
      core_axis_name="c", subcore_axis_name="s",
      num_cores=_NC, num_subcores=_NS)

  @functools.partial(
      pl.kernel,
      out_type=jax.ShapeDtypeStruct((nch * chunk, d), table.dtype),
      mesh=mesh,
      scratch_types=[
          pltpu.VMEM((nk, chunk), jnp.int32),
          pltpu.VMEM((chunk, d), table.dtype),
          pltpu.VMEM((chunk, d), table.dtype),
          pltpu.SemaphoreType.DMA,
          pltpu.SemaphoreType.DMA,
          pltpu.SemaphoreType.DMA,
      ],
  )
  def gather_kernel(table_hbm, idx_hbm, out_hbm, idx_v, r0, r1, sg, so0, so1):
    w = lax.axis_index("s") * _NC + lax.axis_index("c")
    pltpu.sync_copy(idx_hbm.at[w], idx_v)
    if contig:
      vc = jnp.clip(nch - w * nk, 0, nk)
    else:
      vc = (nch - w + _NW - 1) // _NW

    def half(k, rb, so):
      cid = (w * nk + k) if contig else (k * _NW + w)

      @pl.when(k < vc)
      def _():
        @pl.when(k >= 2)
        def _():
          pltpu.make_async_copy(rb, out_hbm.at[pl.ds(0, chunk)], so).wait()

        pltpu.async_copy(table_hbm.at[idx_v.at[k]], rb, sg).wait()
        pltpu.async_copy(rb, out_hbm.at[pl.ds(cid * chunk, chunk)], so)

    def body(j, _):
      half(2 * j, r0, so0)
      half(2 * j + 1, r1, so1)
      return 0

    lax.fori_loop(0, (nk + 1) // 2, body, 0, unroll=False)

    for p, so in ((0, so0), (1, so1)):
      @pl.when((vc + 1 - p) // 2 >= 1)
      def _():
        pltpu.make_async_copy(r0 if p == 0 else r1,
                              out_hbm.at[pl.ds(0, chunk)], so).wait()

  return gather_kernel(table, idx3w)


def _tc_project(nfeat, w):
  def body(nf_ref, w_ref, out_ref):
    out_ref[...] = jnp.dot(nf_ref[...], w_ref[...], precision=_PREC,
                           preferred_element_type=jnp.float32)

  return pl.pallas_call(
      body,
      out_shape=jax.ShapeDtypeStruct((nfeat.shape[0], w.shape[1]), jnp.float32),
  )(nfeat, w)


def _tc_encode(g0, efeat, ts, w_e, w_t, bt, bf, ph, block):
  e = g0.shape[0]
  h = w_t.shape[1]
  grid = e // block

  _c = (1.00000000e+00, -4.99999999e-01, 4.16666642e-02, -1.38888675e-03,
        2.48006914e-05, -2.75369917e-07, 2.06207486e-09, -9.77506520e-12)

  def _fast_cos(x):
    n = jnp.floor(x * jnp.float32(0.15915494309189535) + jnp.float32(0.5))
    r = (x - n * jnp.float32(6.28125)) - n * jnp.float32(0.0019353071795864769)
    s = r * r
    acc = jnp.float32(_c[7])
    for c in _c[6::-1]:
      acc = acc * s + jnp.float32(c)
    return acc

  def body(g_ref, ef_ref, ts_ref, we_ref, wt_ref, bt_ref, bf_ref, ph_ref, out_ref):
    t_col = ts_ref[0].reshape(block, 1)
    te = _fast_cos(t_col * bf_ref[...] + ph_ref[...])
    acc = g_ref[...] + bt_ref[...]
    acc += lax.dot_general(ef_ref[...], we_ref[...],
                           dimension_numbers=(((0,), (0,)), ((), ())),
                           precision=_PREC,
                           preferred_element_type=jnp.float32)
    acc += jnp.dot(te, wt_ref[...], precision=_PREC,
                   preferred_element_type=jnp.float32)
    out_ref[...] = jnp.maximum(acc, 0.0)

  return pl.pallas_call(
      body,
      grid=(grid,),
      in_specs=[
          pl.BlockSpec((block, h), lambda i: (i, 0)),
          pl.BlockSpec((efeat.shape[0], block), lambda i: (0, i)),
          pl.BlockSpec((1, 1, block), lambda i: (i, 0, 0)),
          pl.BlockSpec(w_e.shape, lambda i: (0, 0)),
          pl.BlockSpec(w_t.shape, lambda i: (0, 0)),
          pl.BlockSpec((1, h), lambda i: (0, 0)),
          pl.BlockSpec((1, h), lambda i: (0, 0)),
          pl.BlockSpec((1, h), lambda i: (0, 0)),
      ],
      out_specs=pl.BlockSpec((block, h), lambda i: (i, 0)),
      out_shape=jax.ShapeDtypeStruct((e, h), jnp.float32),
      compiler_params=pltpu.CompilerParams(
          dimension_semantics=("parallel",)),
  )(g0, efeat, ts, w_e, w_t, bt, bf, ph)


def _tc_sage_layer(dst, gsrc, seg_row, ws, wn, b, block):
  e, h = dst.shape
  grid = e // block

  sub = 320
  nsub = block // sub

  def body(dst_ref, gs_ref, sr_ref, ws_ref, wn_ref, b_ref, out_ref,
           carry_sum, carry_cnt, carry_seg):
    @pl.when(pl.program_id(0) == 0)
    def _():
      carry_seg[0] = -1
      carry_cnt[0] = 0.0
      carry_sum[...] = jnp.zeros_like(carry_sum)

    seg_full = sr_ref[0]
    rowid = lax.broadcasted_iota(jnp.int32, (sub, sub), 0)
    colid = lax.broadcasted_iota(jnp.int32, (sub, sub), 1)
    tri = colid <= rowid

    c_sum = carry_sum[...]
    c_cnt = carry_cnt[0]
    c_seg = carry_seg[0]
    aggs = []
    for p in range(nsub):
      seg_r = seg_full[:, p * sub:(p + 1) * sub]
      seg_c = seg_r.reshape(sub, 1)
      mf = (tri & (seg_c == seg_r)).astype(jnp.float32)
      cs = jnp.dot(mf, gs_ref[p * sub:(p + 1) * sub, :], precision=_PREC,
                   preferred_element_type=jnp.float32)
      cntl = jnp.sum(mf, axis=1, keepdims=True)
      fc = (seg_c == c_seg).astype(jnp.float32)
      total = cs + fc * c_sum
      cnt = cntl + fc * c_cnt
      aggs.append(total / cnt)
      c_sum = total[sub - 1:sub, :]
      c_cnt = cnt[sub - 1, 0]
      c_seg = seg_c[sub - 1, 0]

    agg = jnp.concatenate(aggs, axis=0)
    carry_sum[...] = c_sum
    carry_cnt[0] = c_cnt
    carry_seg[0] = c_seg

    acc = jnp.dot(dst_ref[...], ws_ref[...], precision=_PREC,
                  preferred_element_type=jnp.float32)
    acc += jnp.dot(agg, wn_ref[...], precision=_PREC,
                   preferred_element_type=jnp.float32)
    out_ref[...] = jnp.maximum(acc + b_ref[...], 0.0)

  return pl.pallas_call(
      body,
      grid=(grid,),
      in_specs=[
          pl.BlockSpec((block, h), lambda i: (i, 0)),
          pl.BlockSpec((block, h), lambda i: (i, 0)),
          pl.BlockSpec((1, 1, block), lambda i: (i, 0, 0)),
          pl.BlockSpec((h, h), lambda i: (0, 0)),
          pl.BlockSpec((h, h), lambda i: (0, 0)),
          pl.BlockSpec((1, h), lambda i: (0, 0)),
      ],
      out_specs=pl.BlockSpec((block, h), lambda i: (i, 0)),
      out_shape=jax.ShapeDtypeStruct((e, h), jnp.float32),
      scratch_shapes=[
          pltpu.VMEM((1, h), jnp.float32),
          pltpu.SMEM((1,), jnp.float32),
          pltpu.SMEM((1,), jnp.int32),
      ],
      compiler_params=pltpu.CompilerParams(
          dimension_semantics=("arbitrary",)),
  )(dst, gsrc, seg_row, ws, wn, b)


def kernel(nfeat, efeat, timestamp, basis_freq, phase, Wt, bt,
           Ws1, bs1, Wn1, bn1, Ws2, bs2, Wn2, bn2, dst_ids, src_max_eid):
  e = efeat.shape[0]
  f = nfeat.shape[1]
  h = Wt.shape[1]
  ef = efeat.shape[1]

  gchunk = 128
  dst_i = dst_ids.astype(jnp.int32)
  src_i = src_max_eid.astype(jnp.int32)
  nch = e // gchunk
  nk = (nch + _NW - 1) // _NW
  pad = nk * _NW - nch

  def _worker_major(idx):
    idx2d = idx.reshape(nch, gchunk)
    if pad:
      idx2d = jnp.concatenate(
          [idx2d, jnp.zeros((pad, gchunk), jnp.int32)], axis=0)
    return idx2d.reshape(nk, _NW, gchunk).swapaxes(0, 1)

  def _contig_major(idx):
    idx2d = idx.reshape(nch, gchunk)
    if pad:
      idx2d = jnp.concatenate(
          [idx2d, jnp.zeros((pad, gchunk), jnp.int32)], axis=0)
    return idx2d.reshape(_NW, nk, gchunk)

  dst_idx3 = _contig_major(dst_i)
  src_idx3 = _worker_major(src_i)

  wt_u = Wt[:f]
  wt_e = Wt[f:f + ef]
  wt_t = Wt[f + ef:]

  proj = _tc_project(nfeat, wt_u)
  g0 = _sc_gather(proj, dst_idx3, nch, gchunk, contig=True)

  eblk = 3200
  ts3 = timestamp.reshape(e // eblk, 1, eblk)
  ef_t = jnp.swapaxes(efeat, 0, 1)
  dst0 = _tc_encode(g0, ef_t, ts3, wt_e, wt_t,
                    bt.reshape(1, h), basis_freq.reshape(1, h),
                    phase.reshape(1, h), block=eblk)

  sblk = 3200
  seg_row = dst_i.reshape(e // sblk, 1, sblk)

  gsrc0 = _sc_gather(dst0, src_idx3, nch, gchunk)
  dst1 = _tc_sage_layer(dst0, gsrc0, seg_row,
                        Ws1, Wn1, (bs1 + bn1).reshape(1, h), sblk)
  gsrc1 = _sc_gather(dst1, src_idx3, nch, gchunk)
  dst2 = _tc_sage_layer(dst1, gsrc1, seg_row,
                        Ws2, Wn2, (bs2 + bn2).reshape(1, h), sblk)
  src = _sc_gather(dst2, src_idx3, nch, gchunk)
  return (src, dst2)


# speedup vs baseline: 4.3901x; 1.1152x over previous
import functools

import jax
import jax.numpy as jnp
from jax import lax
from jax.experimental import pallas as pl
from jax.experimental.pallas import tpu as pltpu
from jax.experimental.pallas import tpu_sc as plsc

_NC = 2
_NS = 16
_NW = _NC * _NS

_PREC = lax.Precision.DEFAULT


def _sc_gather(table, idx3w, nch, chunk, contig=False):
  nk = idx3w.shape[1]
  d = table.shape[1]
  mesh = plsc.VectorSubcoreMesh(
      core_axis_name="c", subcore_axis_name="s",
      num_cores=_NC, num_subcores=_NS)

  @functools.partial(
      pl.kernel,
      out_type=jax.ShapeDtypeStruct((nch * chunk, d), table.dtype),
      mesh=mesh,
      scratch_types=[
          pltpu.VMEM((nk, chunk), jnp.int32),
          pltpu.VMEM((chunk, d), table.dtype),
          pltpu.VMEM((chunk, d), table.dtype),
          pltpu.SemaphoreType.DMA,
          pltpu.SemaphoreType.DMA,
          pltpu.SemaphoreType.DMA,
          pltpu.SemaphoreType.DMA,
      ],
  )
  def gather_kernel(table_hbm, idx_hbm, out_hbm, idx_v, r0, r1,
                    sg0, sg1, so0, so1):
    w = lax.axis_index("s") * _NC + lax.axis_index("c")
    pltpu.sync_copy(idx_hbm.at[w], idx_v)
    if contig:
      vc = jnp.clip(nch - w * nk, 0, nk)
    else:
      vc = (nch - w + _NW - 1) // _NW

    def stage(k, rb, sg, so):
      @pl.when(k < vc)
      def _():
        @pl.when(k >= 2)
        def _():
          pltpu.make_async_copy(rb, out_hbm.at[pl.ds(0, chunk)], so).wait()

        pltpu.async_copy(table_hbm.at[idx_v.at[k]], rb, sg)

    def drain(k, rb, sg, so):
      cid = (w * nk + k) if contig else (k * _NW + w)

      @pl.when(k < vc)
      def _():
        pltpu.make_async_copy(table_hbm.at[idx_v.at[k]], rb, sg).wait()
        pltpu.async_copy(rb, out_hbm.at[pl.ds(cid * chunk, chunk)], so)

    stage(0, r0, sg0, so0)

    def body(j, _):
      stage(2 * j + 1, r1, sg1, so1)
      drain(2 * j, r0, sg0, so0)
      stage(2 * j + 2, r0, sg0, so0)
      drain(2 * j + 1, r1, sg1, so1)
      return 0

    lax.fori_loop(0, (nk + 1) // 2, body, 0, unroll=False)

    for p, so in ((0, so0), (1, so1)):
      @pl.when((vc + 1 - p) // 2 >= 1)
      def _():
        pltpu.make_async_copy(r0 if p == 0 else r1,
                              out_hbm.at[pl.ds(0, chunk)], so).wait()

  return gather_kernel(table, idx3w)


def _tc_project(nfeat, w):
  def body(nf_ref, w_ref, out_ref):
    out_ref[...] = jnp.dot(nf_ref[...], w_ref[...], precision=_PREC,
                           preferred_element_type=jnp.float32)

  return pl.pallas_call(
      body,
      out_shape=jax.ShapeDtypeStruct((nfeat.shape[0], w.shape[1]), jnp.float32),
  )(nfeat, w)


def _tc_encode(g0, efeat, ts, w_e, w_t, bt, bf, ph, block):
  e = g0.shape[0]
  h = w_t.shape[1]
  grid = e // block

  _c = (1.00000000e+00, -4.99999999e-01, 4.16666642e-02, -1.38888675e-03,
        2.48006914e-05, -2.75369917e-07, 2.06207486e-09, -9.77506520e-12)

  def _fast_cos(x):
    n = jnp.floor(x * jnp.float32(0.15915494309189535) + jnp.float32(0.5))
    r = (x - n * jnp.float32(6.28125)) - n * jnp.float32(0.0019353071795864769)
    s = r * r
    acc = jnp.float32(_c[7])
    for c in _c[6::-1]:
      acc = acc * s + jnp.float32(c)
    return acc

  def body(g_ref, ef_ref, ts_ref, we_ref, wt_ref, bt_ref, bf_ref, ph_ref, out_ref):
    t_col = ts_ref[0].reshape(block, 1)
    te = _fast_cos(t_col * bf_ref[...] + ph_ref[...])
    acc = g_ref[...] + bt_ref[...]
    acc += lax.dot_general(ef_ref[...], we_ref[...],
                           dimension_numbers=(((0,), (0,)), ((), ())),
                           precision=_PREC,
                           preferred_element_type=jnp.float32)
    acc += jnp.dot(te, wt_ref[...], precision=_PREC,
                   preferred_element_type=jnp.float32)
    out_ref[...] = jnp.maximum(acc, 0.0)

  return pl.pallas_call(
      body,
      grid=(grid,),
      in_specs=[
          pl.BlockSpec((block, h), lambda i: (i, 0)),
          pl.BlockSpec((efeat.shape[0], block), lambda i: (0, i)),
          pl.BlockSpec((1, 1, block), lambda i: (i, 0, 0)),
          pl.BlockSpec(w_e.shape, lambda i: (0, 0)),
          pl.BlockSpec(w_t.shape, lambda i: (0, 0)),
          pl.BlockSpec((1, h), lambda i: (0, 0)),
          pl.BlockSpec((1, h), lambda i: (0, 0)),
          pl.BlockSpec((1, h), lambda i: (0, 0)),
      ],
      out_specs=pl.BlockSpec((block, h), lambda i: (i, 0)),
      out_shape=jax.ShapeDtypeStruct((e, h), jnp.float32),
      compiler_params=pltpu.CompilerParams(
          dimension_semantics=("parallel",)),
  )(g0, efeat, ts, w_e, w_t, bt, bf, ph)


def _tc_sage_layer(dst, gsrc, seg_row, ws, wn, b, block):
  e, h = dst.shape
  grid = e // block

  sub = 320
  nsub = block // sub

  def body(dst_ref, gs_ref, sr_ref, ws_ref, wn_ref, b_ref, out_ref,
           carry_sum, carry_cnt, carry_seg):
    @pl.when(pl.program_id(0) == 0)
    def _():
      carry_seg[0] = -1
      carry_cnt[0] = 0.0
      carry_sum[...] = jnp.zeros_like(carry_sum)

    seg_full = sr_ref[0]
    rowid = lax.broadcasted_iota(jnp.int32, (sub, sub), 0)
    colid = lax.broadcasted_iota(jnp.int32, (sub, sub), 1)
    tri = colid <= rowid

    c_sum = carry_sum[...]
    c_cnt = carry_cnt[0]
    c_seg = carry_seg[0]
    aggs = []
    for p in range(nsub):
      seg_r = seg_full[:, p * sub:(p + 1) * sub]
      seg_c = seg_r.reshape(sub, 1)
      mf = (tri & (seg_c == seg_r)).astype(jnp.float32)
      cs = jnp.dot(mf, gs_ref[p * sub:(p + 1) * sub, :], precision=_PREC,
                   preferred_element_type=jnp.float32)
      cntl = jnp.sum(mf, axis=1, keepdims=True)
      fc = (seg_c == c_seg).astype(jnp.float32)
      total = cs + fc * c_sum
      cnt = cntl + fc * c_cnt
      aggs.append(total / cnt)
      c_sum = total[sub - 1:sub, :]
      c_cnt = cnt[sub - 1, 0]
      c_seg = seg_c[sub - 1, 0]

    agg = jnp.concatenate(aggs, axis=0)
    carry_sum[...] = c_sum
    carry_cnt[0] = c_cnt
    carry_seg[0] = c_seg

    acc = jnp.dot(dst_ref[...], ws_ref[...], precision=_PREC,
                  preferred_element_type=jnp.float32)
    acc += jnp.dot(agg, wn_ref[...], precision=_PREC,
                   preferred_element_type=jnp.float32)
    out_ref[...] = jnp.maximum(acc + b_ref[...], 0.0)

  return pl.pallas_call(
      body,
      grid=(grid,),
      in_specs=[
          pl.BlockSpec((block, h), lambda i: (i, 0)),
          pl.BlockSpec((block, h), lambda i: (i, 0)),
          pl.BlockSpec((1, 1, block), lambda i: (i, 0, 0)),
          pl.BlockSpec((h, h), lambda i: (0, 0)),
          pl.BlockSpec((h, h), lambda i: (0, 0)),
          pl.BlockSpec((1, h), lambda i: (0, 0)),
      ],
      out_specs=pl.BlockSpec((block, h), lambda i: (i, 0)),
      out_shape=jax.ShapeDtypeStruct((e, h), jnp.float32),
      scratch_shapes=[
          pltpu.VMEM((1, h), jnp.float32),
          pltpu.SMEM((1,), jnp.float32),
          pltpu.SMEM((1,), jnp.int32),
      ],
      compiler_params=pltpu.CompilerParams(
          dimension_semantics=("arbitrary",)),
  )(dst, gsrc, seg_row, ws, wn, b)


def kernel(nfeat, efeat, timestamp, basis_freq, phase, Wt, bt,
           Ws1, bs1, Wn1, bn1, Ws2, bs2, Wn2, bn2, dst_ids, src_max_eid):
  e = efeat.shape[0]
  f = nfeat.shape[1]
  h = Wt.shape[1]
  ef = efeat.shape[1]

  gchunk = 128
  dst_i = dst_ids.astype(jnp.int32)
  src_i = src_max_eid.astype(jnp.int32)
  nch = e // gchunk
  nk = (nch + _NW - 1) // _NW
  pad = nk * _NW - nch

  def _worker_major(idx):
    idx2d = idx.reshape(nch, gchunk)
    if pad:
      idx2d = jnp.concatenate(
          [idx2d, jnp.zeros((pad, gchunk), jnp.int32)], axis=0)
    return idx2d.reshape(nk, _NW, gchunk).swapaxes(0, 1)

  def _contig_major(idx):
    idx2d = idx.reshape(nch, gchunk)
    if pad:
      idx2d = jnp.concatenate(
          [idx2d, jnp.zeros((pad, gchunk), jnp.int32)], axis=0)
    return idx2d.reshape(_NW, nk, gchunk)

  dst_idx3 = _contig_major(dst_i)
  src_idx3 = _worker_major(src_i)

  wt_u = Wt[:f]
  wt_e = Wt[f:f + ef]
  wt_t = Wt[f + ef:]

  proj = _tc_project(nfeat, wt_u)
  g0 = _sc_gather(proj, dst_idx3, nch, gchunk, contig=True)

  eblk = 3200
  ts3 = timestamp.reshape(e // eblk, 1, eblk)
  ef_t = jnp.swapaxes(efeat, 0, 1)
  dst0 = _tc_encode(g0, ef_t, ts3, wt_e, wt_t,
                    bt.reshape(1, h), basis_freq.reshape(1, h),
                    phase.reshape(1, h), block=eblk)

  sblk = 3200
  seg_row = dst_i.reshape(e // sblk, 1, sblk)

  gsrc0 = _sc_gather(dst0, src_idx3, nch, gchunk)
  dst1 = _tc_sage_layer(dst0, gsrc0, seg_row,
                        Ws1, Wn1, (bs1 + bn1).reshape(1, h), sblk)
  gsrc1 = _sc_gather(dst1, src_idx3, nch, gchunk)
  dst2 = _tc_sage_layer(dst1, gsrc1, seg_row,
                        Ws2, Wn2, (bs2 + bn2).reshape(1, h), sblk)
  src = _sc_gather(dst2, src_idx3, nch, gchunk)
  return (src, dst2)
